# Initial kernel scaffold; baseline (speedup 1.0000x reference)
import functools

import jax
import jax.numpy as jnp
from jax import lax
from jax.experimental import pallas as pl
from jax.experimental.pallas import tpu as pltpu
from jax.experimental.pallas import tpu_sc as plsc

_N = 50000
_E = 800000
_D_IN = 256
_H1 = 128
_D_OUT = 64
_HALF = 32
_NS = 16
_GBLK = 80
_CHUNK_BLKS = 25
_CHUNK = _GBLK * _CHUNK_BLKS
_TILE_EDGES = _E // _NS
_N_CHUNKS = _TILE_EDGES // _CHUNK
_WB_ROWS = 3120
_ROW_BLK = 1000


def _mlp_body(x_ref, w1_ref, b1_ref, w2_ref, b2_ref, o_ref):
    h = jnp.maximum(
        jnp.dot(x_ref[...], w1_ref[...], preferred_element_type=jnp.float32)
        + b1_ref[...], 0.0)
    o_ref[...] = (
        jnp.dot(h, w2_ref[...], preferred_element_type=jnp.float32)
        + b2_ref[...])


def _mlp(seq_a, W1, b1, W2, b2):
    return pl.pallas_call(
        _mlp_body,
        grid=(_N // _ROW_BLK,),
        in_specs=[
            pl.BlockSpec((_ROW_BLK, _D_IN), lambda i: (i, 0)),
            pl.BlockSpec((_D_IN, _H1), lambda i: (0, 0)),
            pl.BlockSpec((1, _H1), lambda i: (0, 0)),
            pl.BlockSpec((_H1, _D_OUT), lambda i: (0, 0)),
            pl.BlockSpec((1, _D_OUT), lambda i: (0, 0)),
        ],
        out_specs=pl.BlockSpec((_ROW_BLK, _D_OUT), lambda i: (i, 0)),
        out_shape=jax.ShapeDtypeStruct((_N, _D_OUT), jnp.float32),
    )(seq_a, W1, b1.reshape(1, _H1), W2, b2.reshape(1, _D_OUT))


def _fuse_body(a_ref, b_ref, o_ref):
    o_ref[...] = (a_ref[...] + b_ref[...]) * 0.5


def _fuse(a, b):
    spec = pl.BlockSpec((_ROW_BLK, _D_OUT), lambda i: (i, 0))
    return pl.pallas_call(
        _fuse_body,
        grid=(_N // _ROW_BLK,),
        in_specs=[spec, spec],
        out_specs=spec,
        out_shape=jax.ShapeDtypeStruct((_N, _D_OUT), jnp.float32),
    )(a, b)


def _spmm_pair(h2, r1, c1, w1, r2, c2, w2):
    mesh = plsc.VectorSubcoreMesh(core_axis_name="c", subcore_axis_name="s")
    out_type = [jax.ShapeDtypeStruct((_N, _D_OUT), jnp.float32)] * 2
    scratch = [
        pltpu.VMEM_SHARED((_N, _HALF), jnp.float32),
        pltpu.VMEM((_CHUNK_BLKS, _GBLK), jnp.int32),
        pltpu.VMEM((_CHUNK_BLKS, _GBLK), jnp.int32),
        pltpu.VMEM((_CHUNK_BLKS, _GBLK), jnp.int32),
        pltpu.VMEM((_CHUNK_BLKS, _GBLK), jnp.float32),
        pltpu.VMEM((_GBLK, _HALF), jnp.float32),
    ]

    @functools.partial(pl.kernel, out_type=out_type, mesh=mesh,
                       scratch_types=scratch)
    def k(h2_h, r1_h, c1_h, w1_h, r2_h, c2_h, w2_h, hp1_h, hp2_h,
          acc, rowbuf, colbuf, idxbuf, wbuf, gbuf):
        c = lax.axis_index("c")
        s = lax.axis_index("s")
        zero16 = jnp.zeros((16,), jnp.float32)
        wb_base = s * _WB_ROWS
        tail_base = _NS * _WB_ROWS

        for phase in range(2):
            r_h = (r1_h, r2_h)[phase]
            c_h = (c1_h, c2_h)[phase]
            w_h = (w1_h, w2_h)[phase]
            hp_h = (hp1_h, hp2_h)[phase]

            for e in range(_GBLK):
                gbuf[e, pl.ds(0, 16)] = zero16
                gbuf[e, pl.ds(16, 16)] = zero16

            def zero_body(i, _):
                pltpu.sync_copy(
                    gbuf, acc.at[pl.ds(wb_base + i * _GBLK, _GBLK), :])
                return 0
            lax.fori_loop(0, _WB_ROWS // _GBLK, zero_body, 0)

            @pl.when(s == 0)
            def _():
                pltpu.sync_copy(gbuf, acc.at[pl.ds(tail_base, _GBLK), :])

            plsc.subcore_barrier()

            def chunk_body(ch, _):
                blk0 = s * (_TILE_EDGES // _GBLK) + ch * _CHUNK_BLKS
                pltpu.sync_copy(r_h.at[pl.ds(blk0, _CHUNK_BLKS), :], rowbuf)
                pltpu.sync_copy(c_h.at[pl.ds(blk0, _CHUNK_BLKS), :], colbuf)
                pltpu.sync_copy(w_h.at[pl.ds(blk0, _CHUNK_BLKS), :], wbuf)
                for j in range(_CHUNK_BLKS):
                    for q in range(_GBLK // 16):
                        v = colbuf[j, pl.ds(q * 16, 16)]
                        idxbuf[j, pl.ds(q * 16, 16)] = v + v + c

                def blk_body(j, _):
                    pltpu.sync_copy(h2_h.at[idxbuf.at[j]], gbuf)
                    for e in range(_GBLK):
                        ws = wbuf[j, e]
                        gbuf[e, pl.ds(0, 16)] = gbuf[e, pl.ds(0, 16)] * ws
                        gbuf[e, pl.ds(16, 16)] = gbuf[e, pl.ds(16, 16)] * ws
                    pltpu.sync_copy(gbuf, acc.at[rowbuf.at[j]], add=True)
                    return 0
                lax.fori_loop(0, _CHUNK_BLKS, blk_body, 0)
                return 0
            lax.fori_loop(0, _N_CHUNKS, chunk_body, 0)

            plsc.subcore_barrier()

            pltpu.sync_copy(
                acc.at[pl.ds(wb_base, _WB_ROWS), :],
                hp_h.at[pl.ds(wb_base, _WB_ROWS), pl.ds(c * _HALF, _HALF)])

            @pl.when(s == 0)
            def _():
                pltpu.sync_copy(
                    acc.at[pl.ds(tail_base, _GBLK), :],
                    hp_h.at[pl.ds(tail_base, _GBLK), pl.ds(c * _HALF, _HALF)])

            plsc.subcore_barrier()

    return k(h2, r1, c1, w1, r2, c2, w2)


def kernel(seq_a, edge_index1, edge_weight1, edge_index2, edge_weight2,
           W1, b1, W2, b2):
    h_a = _mlp(seq_a, W1, b1, W2, b2)
    h2 = h_a.reshape(2 * _N, _HALF)
    r1 = edge_index1[0].reshape(_E // _GBLK, _GBLK)
    c1 = edge_index1[1].reshape(_E // _GBLK, _GBLK)
    w1 = edge_weight1.reshape(_E // _GBLK, _GBLK)
    r2 = edge_index2[0].reshape(_E // _GBLK, _GBLK)
    c2 = edge_index2[1].reshape(_E // _GBLK, _GBLK)
    w2 = edge_weight2.reshape(_E // _GBLK, _GBLK)
    h_p1, h_p2 = _spmm_pair(h2, r1, c1, w1, r2, c2, w2)
    h_p_fusion = _fuse(h_p1, h_p2)
    return (h_a, h_p1, h_p2, h_p_fusion)


# speedup vs baseline: 5.1059x; 5.1059x over previous
import functools

import jax
import jax.numpy as jnp
from jax import lax
from jax.experimental import pallas as pl
from jax.experimental.pallas import tpu as pltpu
from jax.experimental.pallas import tpu_sc as plsc

_N = 50000
_E = 800000
_D_IN = 256
_H1 = 128
_D_OUT = 64
_HALF = 32
_NS = 16
_GBLK = 128
_CHUNK_BLKS = 8
_E_PAD = 802816
_N_CHUNKS_TOT = _E_PAD // (_GBLK * _CHUNK_BLKS)
_N_CHUNKS = _N_CHUNKS_TOT // _NS
_ZBLK = 80
_WB_ROWS = 3120
_ROW_BLK = 1000


def _mlp_body(x_ref, w1_ref, b1_ref, w2_ref, b2_ref, o_ref):
    h = jnp.maximum(
        jnp.dot(x_ref[...], w1_ref[...], preferred_element_type=jnp.float32)
        + b1_ref[...], 0.0)
    o_ref[...] = (
        jnp.dot(h, w2_ref[...], preferred_element_type=jnp.float32)
        + b2_ref[...])


def _mlp(seq_a, W1, b1, W2, b2):
    return pl.pallas_call(
        _mlp_body,
        grid=(_N // _ROW_BLK,),
        in_specs=[
            pl.BlockSpec((_ROW_BLK, _D_IN), lambda i: (i, 0)),
            pl.BlockSpec((_D_IN, _H1), lambda i: (0, 0)),
            pl.BlockSpec((1, _H1), lambda i: (0, 0)),
            pl.BlockSpec((_H1, _D_OUT), lambda i: (0, 0)),
            pl.BlockSpec((1, _D_OUT), lambda i: (0, 0)),
        ],
        out_specs=pl.BlockSpec((_ROW_BLK, _D_OUT), lambda i: (i, 0)),
        out_shape=jax.ShapeDtypeStruct((_N, _D_OUT), jnp.float32),
    )(seq_a, W1, b1.reshape(1, _H1), W2, b2.reshape(1, _D_OUT))


def _fuse_body(a_ref, b_ref, o_ref):
    o_ref[...] = (a_ref[...] + b_ref[...]) * 0.5


def _fuse(a, b):
    spec = pl.BlockSpec((_ROW_BLK, _D_OUT), lambda i: (i, 0))
    return pl.pallas_call(
        _fuse_body,
        grid=(_N // _ROW_BLK,),
        in_specs=[spec, spec],
        out_specs=spec,
        out_shape=jax.ShapeDtypeStruct((_N, _D_OUT), jnp.float32),
    )(a, b)


def _spmm_pair(h2, r1, c1, w1, r2, c2, w2):
    mesh = plsc.VectorSubcoreMesh(core_axis_name="c", subcore_axis_name="s")
    out_type = [jax.ShapeDtypeStruct((2, _N, _HALF), jnp.float32)] * 2
    scratch = [
        pltpu.VMEM_SHARED((_N, _HALF), jnp.float32),
        pltpu.VMEM((_CHUNK_BLKS, _GBLK), jnp.int32),
        pltpu.VMEM((_CHUNK_BLKS, _GBLK), jnp.int32),
        pltpu.VMEM((_CHUNK_BLKS, _GBLK), jnp.int32),
        pltpu.VMEM((_CHUNK_BLKS, _GBLK), jnp.float32),
        pltpu.VMEM((_GBLK, _HALF), jnp.float32),
    ]

    @functools.partial(
        pl.kernel, out_type=out_type, mesh=mesh, scratch_types=scratch,
        compiler_params=pltpu.CompilerParams(use_tc_tiling_on_sc=False))
    def k(h2_h, r1_h, c1_h, w1_h, r2_h, c2_h, w2_h, hp1_h, hp2_h,
          acc, rowbuf, colbuf, idxbuf, wbuf, gbuf):
        c = lax.axis_index("c")
        s = lax.axis_index("s")
        zero16 = jnp.zeros((16,), jnp.float32)
        wb_base = s * _WB_ROWS
        tail_base = _NS * _WB_ROWS

        for phase in range(2):
            r_h = (r1_h, r2_h)[phase]
            c_h = (c1_h, c2_h)[phase]
            w_h = (w1_h, w2_h)[phase]
            hp_h = (hp1_h, hp2_h)[phase]

            for e in range(_ZBLK):
                gbuf[e, pl.ds(0, 16)] = zero16
                gbuf[e, pl.ds(16, 16)] = zero16

            def zero_body(i, _):
                pltpu.sync_copy(
                    gbuf.at[pl.ds(0, _ZBLK), :],
                    acc.at[pl.ds(wb_base + i * _ZBLK, _ZBLK), :])
                return 0
            lax.fori_loop(0, _WB_ROWS // _ZBLK, zero_body, 0)

            @pl.when(s == 0)
            def _():
                pltpu.sync_copy(gbuf.at[pl.ds(0, _ZBLK), :],
                                acc.at[pl.ds(tail_base, _ZBLK), :])

            plsc.subcore_barrier()

            def chunk_body(ch, _):
                cid = s * _N_CHUNKS + ch
                pltpu.sync_copy(r_h.at[cid], rowbuf)
                pltpu.sync_copy(c_h.at[cid], colbuf)
                pltpu.sync_copy(w_h.at[cid], wbuf)
                for j in range(_CHUNK_BLKS):
                    for q in range(_GBLK // 16):
                        v = colbuf[j, pl.ds(q * 16, 16)]
                        idxbuf[j, pl.ds(q * 16, 16)] = v + v + c

                def blk_body(j, _):
                    pltpu.sync_copy(h2_h.at[idxbuf.at[j]], gbuf)
                    for g in range(_GBLK // 16):
                        wv = wbuf[j, pl.ds(g * 16, 16)]
                        for t in range(16):
                            e = g * 16 + t
                            ws = wv[t]
                            gbuf[e, pl.ds(0, 16)] = gbuf[e, pl.ds(0, 16)] * ws
                            gbuf[e, pl.ds(16, 16)] = gbuf[e, pl.ds(16, 16)] * ws
                    pltpu.sync_copy(gbuf, acc.at[rowbuf.at[j]], add=True)
                    return 0
                lax.fori_loop(0, _CHUNK_BLKS, blk_body, 0)
                return 0
            lax.fori_loop(0, _N_CHUNKS, chunk_body, 0)

            plsc.subcore_barrier()

            pltpu.sync_copy(
                acc.at[pl.ds(wb_base, _WB_ROWS), :],
                hp_h.at[c, pl.ds(wb_base, _WB_ROWS), :])

            @pl.when(s == 0)
            def _():
                pltpu.sync_copy(
                    acc.at[pl.ds(tail_base, _ZBLK), :],
                    hp_h.at[c, pl.ds(tail_base, _ZBLK), :])

            plsc.subcore_barrier()

    return k(h2, r1, c1, w1, r2, c2, w2)


def kernel(seq_a, edge_index1, edge_weight1, edge_index2, edge_weight2,
           W1, b1, W2, b2):
    h_a = _mlp(seq_a, W1, b1, W2, b2)
    h2 = h_a.reshape(2 * _N, _HALF)
    pad = _E_PAD - _E

    def _shape_idx(x):
        return jnp.pad(x, (0, pad)).reshape(
            _N_CHUNKS_TOT, _CHUNK_BLKS, _GBLK)

    r1 = _shape_idx(edge_index1[0])
    c1 = _shape_idx(edge_index1[1])
    w1 = _shape_idx(edge_weight1)
    r2 = _shape_idx(edge_index2[0])
    c2 = _shape_idx(edge_index2[1])
    w2 = _shape_idx(edge_weight2)
    o1, o2 = _spmm_pair(h2, r1, c1, w1, r2, c2, w2)
    h_p1 = jnp.moveaxis(o1, 0, 1).reshape(_N, _D_OUT)
    h_p2 = jnp.moveaxis(o2, 0, 1).reshape(_N, _D_OUT)
    h_p_fusion = _fuse(h_p1, h_p2)
    return (h_a, h_p1, h_p2, h_p_fusion)


# speedup vs baseline: 6.1421x; 1.2029x over previous
import functools

import jax
import jax.numpy as jnp
from jax import lax
from jax.experimental import pallas as pl
from jax.experimental.pallas import tpu as pltpu
from jax.experimental.pallas import tpu_sc as plsc

_N = 50000
_E = 800000
_D_IN = 256
_H1 = 128
_D_OUT = 64
_HALF = 32
_NS = 16
_GBLK = 128
_CHUNK_BLKS = 8
_E_PAD = 802816
_N_CHUNKS_TOT = _E_PAD // (_GBLK * _CHUNK_BLKS)
_N_CHUNKS = _N_CHUNKS_TOT // _NS
_ZBLK = 80
_WB_ROWS = 3120
_ROW_BLK = 1000


def _mlp_body(x_ref, w1_ref, b1_ref, w2_ref, b2_ref, o_ref):
    h = jnp.maximum(
        jnp.dot(x_ref[...], w1_ref[...], preferred_element_type=jnp.float32)
        + b1_ref[...], 0.0)
    o_ref[...] = (
        jnp.dot(h, w2_ref[...], preferred_element_type=jnp.float32)
        + b2_ref[...])


def _mlp(seq_a, W1, b1, W2, b2):
    return pl.pallas_call(
        _mlp_body,
        grid=(_N // _ROW_BLK,),
        in_specs=[
            pl.BlockSpec((_ROW_BLK, _D_IN), lambda i: (i, 0)),
            pl.BlockSpec((_D_IN, _H1), lambda i: (0, 0)),
            pl.BlockSpec((1, _H1), lambda i: (0, 0)),
            pl.BlockSpec((_H1, _D_OUT), lambda i: (0, 0)),
            pl.BlockSpec((1, _D_OUT), lambda i: (0, 0)),
        ],
        out_specs=pl.BlockSpec((_ROW_BLK, _D_OUT), lambda i: (i, 0)),
        out_shape=jax.ShapeDtypeStruct((_N, _D_OUT), jnp.float32),
    )(seq_a, W1, b1.reshape(1, _H1), W2, b2.reshape(1, _D_OUT))


def _fuse_body(a_ref, b_ref, o_ref):
    o_ref[...] = (a_ref[...] + b_ref[...]) * 0.5


def _fuse(a, b):
    spec = pl.BlockSpec((_ROW_BLK, _D_OUT), lambda i: (i, 0))
    return pl.pallas_call(
        _fuse_body,
        grid=(_N // _ROW_BLK,),
        in_specs=[spec, spec],
        out_specs=spec,
        out_shape=jax.ShapeDtypeStruct((_N, _D_OUT), jnp.float32),
    )(a, b)


def _spmm_pair(h2, r1, c1, w1, r2, c2, w2):
    mesh = plsc.VectorSubcoreMesh(core_axis_name="c", subcore_axis_name="s")
    out_type = [jax.ShapeDtypeStruct((2, _N, _HALF), jnp.float32)] * 2
    scratch = [
        pltpu.VMEM_SHARED((_N, _HALF), jnp.float32),
        pltpu.VMEM((_CHUNK_BLKS, _GBLK), jnp.int32),
        pltpu.VMEM((_CHUNK_BLKS, _GBLK), jnp.int32),
        pltpu.VMEM((_CHUNK_BLKS, _GBLK), jnp.int32),
        pltpu.VMEM((_CHUNK_BLKS, _GBLK), jnp.float32),
        pltpu.VMEM((2, _GBLK, _HALF), jnp.float32),
        pltpu.SemaphoreType.DMA((2,)),
        pltpu.SemaphoreType.DMA((2,)),
    ]

    @functools.partial(
        pl.kernel, out_type=out_type, mesh=mesh, scratch_types=scratch,
        compiler_params=pltpu.CompilerParams(use_tc_tiling_on_sc=False))
    def k(h2_h, r1_h, c1_h, w1_h, r2_h, c2_h, w2_h, hp1_h, hp2_h,
          acc, rowbuf, colbuf, idxbuf, wbuf, gbuf, gsem, ssem):
        c = lax.axis_index("c")
        s = lax.axis_index("s")
        zero16 = jnp.zeros((16,), jnp.float32)
        wb_base = s * _WB_ROWS
        tail_base = _NS * _WB_ROWS

        for phase in range(2):
            r_h = (r1_h, r2_h)[phase]
            c_h = (c1_h, c2_h)[phase]
            w_h = (w1_h, w2_h)[phase]
            hp_h = (hp1_h, hp2_h)[phase]

            for e in range(_ZBLK):
                gbuf[0, e, pl.ds(0, 16)] = zero16
                gbuf[0, e, pl.ds(16, 16)] = zero16

            def zero_body(i, _):
                pltpu.sync_copy(
                    gbuf.at[0, pl.ds(0, _ZBLK), :],
                    acc.at[pl.ds(wb_base + i * _ZBLK, _ZBLK), :])
                return 0
            lax.fori_loop(0, _WB_ROWS // _ZBLK, zero_body, 0)

            @pl.when(s == 0)
            def _():
                pltpu.sync_copy(gbuf.at[0, pl.ds(0, _ZBLK), :],
                                acc.at[pl.ds(tail_base, _ZBLK), :])

            plsc.subcore_barrier()

            def start_gather(j):
                pltpu.async_copy(
                    h2_h.at[idxbuf.at[j]], gbuf.at[j % 2], gsem.at[j % 2])

            def wait_gather(j):
                pltpu.make_async_copy(
                    h2_h.at[idxbuf.at[j]], gbuf.at[j % 2],
                    gsem.at[j % 2]).wait()

            def start_scatter(j):
                pltpu.async_copy(
                    gbuf.at[j % 2], acc.at[rowbuf.at[j]], ssem.at[j % 2],
                    add=True)

            def wait_scatter(j):
                pltpu.make_async_copy(
                    gbuf.at[j % 2], acc.at[rowbuf.at[j]],
                    ssem.at[j % 2]).wait()

            def chunk_body(ch, _):
                cid = s * _N_CHUNKS + ch
                pltpu.sync_copy(r_h.at[cid], rowbuf)
                pltpu.sync_copy(c_h.at[cid], colbuf)
                pltpu.sync_copy(w_h.at[cid], wbuf)
                for j in range(_CHUNK_BLKS):
                    for q in range(_GBLK // 16):
                        v = colbuf[j, pl.ds(q * 16, 16)]
                        idxbuf[j, pl.ds(q * 16, 16)] = v + v + c

                start_gather(0)

                def blk_body(j, _):
                    j2 = j % 2
                    wait_gather(j)

                    @pl.when(j < _CHUNK_BLKS - 1)
                    def _():
                        start_gather(j + 1)

                    @pl.when(j >= 2)
                    def _():
                        wait_scatter(j - 2)

                    for g in range(_GBLK // 16):
                        wv = wbuf[j, pl.ds(g * 16, 16)]
                        for t in range(16):
                            e = g * 16 + t
                            ws = wv[t]
                            gbuf[j2, e, pl.ds(0, 16)] = (
                                gbuf[j2, e, pl.ds(0, 16)] * ws)
                            gbuf[j2, e, pl.ds(16, 16)] = (
                                gbuf[j2, e, pl.ds(16, 16)] * ws)
                    start_scatter(j)
                    return 0
                lax.fori_loop(0, _CHUNK_BLKS, blk_body, 0)
                wait_scatter(_CHUNK_BLKS - 2)
                wait_scatter(_CHUNK_BLKS - 1)
                return 0
            lax.fori_loop(0, _N_CHUNKS, chunk_body, 0)

            plsc.subcore_barrier()

            pltpu.sync_copy(
                acc.at[pl.ds(wb_base, _WB_ROWS), :],
                hp_h.at[c, pl.ds(wb_base, _WB_ROWS), :])

            @pl.when(s == 0)
            def _():
                pltpu.sync_copy(
                    acc.at[pl.ds(tail_base, _ZBLK), :],
                    hp_h.at[c, pl.ds(tail_base, _ZBLK), :])

            plsc.subcore_barrier()

    return k(h2, r1, c1, w1, r2, c2, w2)


def kernel(seq_a, edge_index1, edge_weight1, edge_index2, edge_weight2,
           W1, b1, W2, b2):
    h_a = _mlp(seq_a, W1, b1, W2, b2)
    h2 = h_a.reshape(2 * _N, _HALF)
    pad = _E_PAD - _E

    def _shape_idx(x):
        return jnp.pad(x, (0, pad)).reshape(
            _N_CHUNKS_TOT, _CHUNK_BLKS, _GBLK)

    r1 = _shape_idx(edge_index1[0])
    c1 = _shape_idx(edge_index1[1])
    w1 = _shape_idx(edge_weight1)
    r2 = _shape_idx(edge_index2[0])
    c2 = _shape_idx(edge_index2[1])
    w2 = _shape_idx(edge_weight2)
    o1, o2 = _spmm_pair(h2, r1, c1, w1, r2, c2, w2)
    h_p1 = jnp.moveaxis(o1, 0, 1).reshape(_N, _D_OUT)
    h_p2 = jnp.moveaxis(o2, 0, 1).reshape(_N, _D_OUT)
    h_p_fusion = _fuse(h_p1, h_p2)
    return (h_a, h_p1, h_p2, h_p_fusion)


# speedup vs baseline: 9.3117x; 1.5160x over previous
import functools

import jax
import jax.numpy as jnp
from jax import lax
from jax.experimental import pallas as pl
from jax.experimental.pallas import tpu as pltpu
from jax.experimental.pallas import tpu_sc as plsc

_N = 50000
_E = 800000
_D_IN = 256
_H1 = 128
_D_OUT = 64
_HALF = 32
_NS = 16
_GBLK = 128
_CHUNK_BLKS = 8
_E_PAD = 802816
_N_CHUNKS_TOT = _E_PAD // (_GBLK * _CHUNK_BLKS)
_N_CHUNKS = _N_CHUNKS_TOT // _NS
_ZBLK = 80
_NB = 4
_WB_ROWS = 3120
_ROW_BLK = 1000


def _mlp_body(x_ref, w1_ref, b1_ref, w2_ref, b2_ref, o_ref):
    h = jnp.maximum(
        jnp.dot(x_ref[...], w1_ref[...], preferred_element_type=jnp.float32)
        + b1_ref[...], 0.0)
    o_ref[...] = (
        jnp.dot(h, w2_ref[...], preferred_element_type=jnp.float32)
        + b2_ref[...])


def _mlp(seq_a, W1, b1, W2, b2):
    return pl.pallas_call(
        _mlp_body,
        grid=(_N // _ROW_BLK,),
        in_specs=[
            pl.BlockSpec((_ROW_BLK, _D_IN), lambda i: (i, 0)),
            pl.BlockSpec((_D_IN, _H1), lambda i: (0, 0)),
            pl.BlockSpec((1, _H1), lambda i: (0, 0)),
            pl.BlockSpec((_H1, _D_OUT), lambda i: (0, 0)),
            pl.BlockSpec((1, _D_OUT), lambda i: (0, 0)),
        ],
        out_specs=pl.BlockSpec((_ROW_BLK, _D_OUT), lambda i: (i, 0)),
        out_shape=jax.ShapeDtypeStruct((_N, _D_OUT), jnp.float32),
    )(seq_a, W1, b1.reshape(1, _H1), W2, b2.reshape(1, _D_OUT))


def _fuse_body(a_ref, b_ref, o_ref):
    o_ref[...] = (a_ref[...] + b_ref[...]) * 0.5


def _fuse(a, b):
    spec = pl.BlockSpec((_ROW_BLK, _D_OUT), lambda i: (i, 0))
    return pl.pallas_call(
        _fuse_body,
        grid=(_N // _ROW_BLK,),
        in_specs=[spec, spec],
        out_specs=spec,
        out_shape=jax.ShapeDtypeStruct((_N, _D_OUT), jnp.float32),
    )(a, b)


def _spmm_pair(h2, r1, c1, w1, r2, c2, w2):
    mesh = plsc.VectorSubcoreMesh(core_axis_name="c", subcore_axis_name="s")
    out_type = [jax.ShapeDtypeStruct((2, _N, _HALF), jnp.float32)] * 2
    scratch = [
        pltpu.VMEM_SHARED((_N, _HALF), jnp.float32),
        pltpu.VMEM((2, _CHUNK_BLKS, _GBLK), jnp.int32),
        pltpu.VMEM((2, _CHUNK_BLKS, _GBLK), jnp.int32),
        pltpu.VMEM((2, _CHUNK_BLKS, _GBLK), jnp.int32),
        pltpu.VMEM((2, _CHUNK_BLKS, _GBLK), jnp.float32),
        pltpu.VMEM((_NB, _GBLK, _HALF), jnp.float32),
        pltpu.SemaphoreType.DMA((_NB,)),
        pltpu.SemaphoreType.DMA((_NB,)),
        pltpu.SemaphoreType.DMA((2,)),
    ]

    @functools.partial(
        pl.kernel, out_type=out_type, mesh=mesh, scratch_types=scratch,
        compiler_params=pltpu.CompilerParams(use_tc_tiling_on_sc=False))
    def k(h2_h, r1_h, c1_h, w1_h, r2_h, c2_h, w2_h, hp1_h, hp2_h,
          acc, rowbuf, colbuf, idxbuf, wbuf, gbuf, gsem, ssem, esem):
        c = lax.axis_index("c")
        s = lax.axis_index("s")
        zero16 = jnp.zeros((16,), jnp.float32)
        wb_base = s * _WB_ROWS
        tail_base = _NS * _WB_ROWS

        for phase in range(2):
            r_h = (r1_h, r2_h)[phase]
            c_h = (c1_h, c2_h)[phase]
            w_h = (w1_h, w2_h)[phase]
            hp_h = (hp1_h, hp2_h)[phase]

            for e in range(_ZBLK):
                gbuf[0, e, pl.ds(0, 16)] = zero16
                gbuf[0, e, pl.ds(16, 16)] = zero16

            def zero_body(i, _):
                pltpu.async_copy(
                    gbuf.at[0, pl.ds(0, _ZBLK), :],
                    acc.at[pl.ds(wb_base + i * _ZBLK, _ZBLK), :],
                    esem.at[0])
                return 0
            lax.fori_loop(0, _WB_ROWS // _ZBLK, zero_body, 0)

            @pl.when(s == 0)
            def _():
                pltpu.async_copy(gbuf.at[0, pl.ds(0, _ZBLK), :],
                                 acc.at[pl.ds(tail_base, _ZBLK), :],
                                 esem.at[0])

            def zero_drain(i, _):
                pltpu.make_async_copy(
                    gbuf.at[0, pl.ds(0, _ZBLK), :],
                    acc.at[pl.ds(wb_base + i * _ZBLK, _ZBLK), :],
                    esem.at[0]).wait()
                return 0
            lax.fori_loop(0, _WB_ROWS // _ZBLK, zero_drain, 0)

            @pl.when(s == 0)
            def _():
                pltpu.make_async_copy(
                    gbuf.at[0, pl.ds(0, _ZBLK), :],
                    acc.at[pl.ds(tail_base, _ZBLK), :],
                    esem.at[0]).wait()

            plsc.subcore_barrier()

            def start_edges(ch):
                cb = ch % 2
                cid = s * _N_CHUNKS + ch
                pltpu.async_copy(r_h.at[cid], rowbuf.at[cb], esem.at[cb])
                pltpu.async_copy(c_h.at[cid], colbuf.at[cb], esem.at[cb])
                pltpu.async_copy(w_h.at[cid], wbuf.at[cb], esem.at[cb])

            def wait_edges(ch):
                cb = ch % 2
                cid = s * _N_CHUNKS + ch
                pltpu.make_async_copy(
                    r_h.at[cid], rowbuf.at[cb], esem.at[cb]).wait()
                pltpu.make_async_copy(
                    c_h.at[cid], colbuf.at[cb], esem.at[cb]).wait()
                pltpu.make_async_copy(
                    w_h.at[cid], wbuf.at[cb], esem.at[cb]).wait()

            def start_gather(cb, j):
                pltpu.async_copy(
                    h2_h.at[idxbuf.at[cb, j]], gbuf.at[j % _NB],
                    gsem.at[j % _NB])

            def wait_gather(cb, j):
                pltpu.make_async_copy(
                    h2_h.at[idxbuf.at[cb, j]], gbuf.at[j % _NB],
                    gsem.at[j % _NB]).wait()

            def start_scatter(cb, j):
                pltpu.async_copy(
                    gbuf.at[j % _NB], acc.at[rowbuf.at[cb, j]],
                    ssem.at[j % _NB], add=True)

            def wait_scatter(cb, j):
                pltpu.make_async_copy(
                    gbuf.at[j % _NB], acc.at[rowbuf.at[cb, j]],
                    ssem.at[j % _NB]).wait()

            start_edges(0)

            def chunk_body(ch, _):
                cb = ch % 2
                wait_edges(ch)

                @pl.when(ch < _N_CHUNKS - 1)
                def _():
                    start_edges(ch + 1)

                for j in range(_CHUNK_BLKS):
                    for q in range(_GBLK // 16):
                        v = colbuf[cb, j, pl.ds(q * 16, 16)]
                        idxbuf[cb, j, pl.ds(q * 16, 16)] = v + v + c

                start_gather(cb, 0)
                start_gather(cb, 1)
                start_gather(cb, 2)

                def blk_body(j, _):
                    jb = j % _NB
                    wait_gather(cb, j)

                    @pl.when(j < _CHUNK_BLKS - _NB + 1)
                    def _():
                        start_gather(cb, j + _NB - 1)

                    @pl.when(j >= _NB)
                    def _():
                        wait_scatter(cb, j - _NB)

                    for g in range(_GBLK // 16):
                        wv = wbuf[cb, j, pl.ds(g * 16, 16)]
                        for t in range(16):
                            e = g * 16 + t
                            ws = wv[t]
                            gbuf[jb, e, pl.ds(0, 16)] = (
                                gbuf[jb, e, pl.ds(0, 16)] * ws)
                            gbuf[jb, e, pl.ds(16, 16)] = (
                                gbuf[jb, e, pl.ds(16, 16)] * ws)
                    start_scatter(cb, j)
                    return 0
                lax.fori_loop(0, _CHUNK_BLKS, blk_body, 0)
                for j in range(_CHUNK_BLKS - _NB, _CHUNK_BLKS):
                    wait_scatter(cb, j)
                return 0
            lax.fori_loop(0, _N_CHUNKS, chunk_body, 0)

            plsc.subcore_barrier()

            pltpu.sync_copy(
                acc.at[pl.ds(wb_base, _WB_ROWS), :],
                hp_h.at[c, pl.ds(wb_base, _WB_ROWS), :])

            @pl.when(s == 0)
            def _():
                pltpu.sync_copy(
                    acc.at[pl.ds(tail_base, _ZBLK), :],
                    hp_h.at[c, pl.ds(tail_base, _ZBLK), :])

            plsc.subcore_barrier()

    return k(h2, r1, c1, w1, r2, c2, w2)


def kernel(seq_a, edge_index1, edge_weight1, edge_index2, edge_weight2,
           W1, b1, W2, b2):
    h_a = _mlp(seq_a, W1, b1, W2, b2)
    h2 = h_a.reshape(2 * _N, _HALF)
    pad = _E_PAD - _E

    def _shape_idx(x):
        return jnp.pad(x, (0, pad)).reshape(
            _N_CHUNKS_TOT, _CHUNK_BLKS, _GBLK)

    r1 = _shape_idx(edge_index1[0])
    c1 = _shape_idx(edge_index1[1])
    w1 = _shape_idx(edge_weight1)
    r2 = _shape_idx(edge_index2[0])
    c2 = _shape_idx(edge_index2[1])
    w2 = _shape_idx(edge_weight2)
    o1, o2 = _spmm_pair(h2, r1, c1, w1, r2, c2, w2)
    h_p1 = jnp.moveaxis(o1, 0, 1).reshape(_N, _D_OUT)
    h_p2 = jnp.moveaxis(o2, 0, 1).reshape(_N, _D_OUT)
    h_p_fusion = _fuse(h_p1, h_p2)
    return (h_a, h_p1, h_p2, h_p_fusion)


# speedup vs baseline: 9.9421x; 1.0677x over previous
import functools

import jax
import jax.numpy as jnp
from jax import lax
from jax.experimental import pallas as pl
from jax.experimental.pallas import tpu as pltpu
from jax.experimental.pallas import tpu_sc as plsc

_N = 50000
_E = 800000
_D_IN = 256
_H1 = 128
_D_OUT = 64
_HALF = 32
_NS = 16
_GBLK = 128
_CHUNK_BLKS = 8
_E_PAD = 802816
_N_CHUNKS_TOT = _E_PAD // (_GBLK * _CHUNK_BLKS)
_N_CHUNKS = _N_CHUNKS_TOT // _NS
_ZBLK = 80
_NB = 4
_WB_ROWS = 3120
_ROW_BLK = 1000


def _mlp_body(x_ref, w1_ref, b1_ref, w2_ref, b2_ref, o_ref):
    h = jnp.maximum(
        jnp.dot(x_ref[...], w1_ref[...], preferred_element_type=jnp.float32)
        + b1_ref[...], 0.0)
    o_ref[...] = (
        jnp.dot(h, w2_ref[...], preferred_element_type=jnp.float32)
        + b2_ref[...])


def _mlp(seq_a, W1, b1, W2, b2):
    return pl.pallas_call(
        _mlp_body,
        grid=(_N // _ROW_BLK,),
        in_specs=[
            pl.BlockSpec((_ROW_BLK, _D_IN), lambda i: (i, 0)),
            pl.BlockSpec((_D_IN, _H1), lambda i: (0, 0)),
            pl.BlockSpec((1, _H1), lambda i: (0, 0)),
            pl.BlockSpec((_H1, _D_OUT), lambda i: (0, 0)),
            pl.BlockSpec((1, _D_OUT), lambda i: (0, 0)),
        ],
        out_specs=pl.BlockSpec((_ROW_BLK, _D_OUT), lambda i: (i, 0)),
        out_shape=jax.ShapeDtypeStruct((_N, _D_OUT), jnp.float32),
    )(seq_a, W1, b1.reshape(1, _H1), W2, b2.reshape(1, _D_OUT))


def _fuse_body(a_ref, b_ref, o_ref):
    o_ref[...] = (a_ref[...] + b_ref[...]) * 0.5


def _fuse(a, b):
    spec = pl.BlockSpec((_ROW_BLK, _D_OUT), lambda i: (i, 0))
    return pl.pallas_call(
        _fuse_body,
        grid=(_N // _ROW_BLK,),
        in_specs=[spec, spec],
        out_specs=spec,
        out_shape=jax.ShapeDtypeStruct((_N, _D_OUT), jnp.float32),
    )(a, b)


def _spmm_pair(h2, r1, c1, w1, r2, c2, w2):
    mesh = plsc.VectorSubcoreMesh(core_axis_name="c", subcore_axis_name="s")
    out_type = [jax.ShapeDtypeStruct((_N, _D_OUT), jnp.float32)] * 2
    scratch = [
        pltpu.VMEM_SHARED((_N, _HALF), jnp.float32),
        pltpu.VMEM((2, _CHUNK_BLKS, _GBLK), jnp.int32),
        pltpu.VMEM((2, _CHUNK_BLKS, _GBLK), jnp.int32),
        pltpu.VMEM((2, _CHUNK_BLKS, _GBLK), jnp.int32),
        pltpu.VMEM((2, _CHUNK_BLKS, _GBLK), jnp.float32),
        pltpu.VMEM((_NB, _GBLK, _HALF), jnp.float32),
        pltpu.SemaphoreType.DMA((_NB,)),
        pltpu.SemaphoreType.DMA((_NB,)),
        pltpu.SemaphoreType.DMA((2,)),
    ]

    @functools.partial(
        pl.kernel, out_type=out_type, mesh=mesh, scratch_types=scratch,
        compiler_params=pltpu.CompilerParams(use_tc_tiling_on_sc=False))
    def k(h2_h, r1_h, c1_h, w1_h, r2_h, c2_h, w2_h, hp1_h, hp2_h,
          acc, rowbuf, colbuf, idxbuf, wbuf, gbuf, gsem, ssem, esem):
        c = lax.axis_index("c")
        s = lax.axis_index("s")
        zero16 = jnp.zeros((16,), jnp.float32)
        wb_base = s * _WB_ROWS
        tail_base = _NS * _WB_ROWS

        for phase in range(2):
            r_h = (r1_h, r2_h)[phase]
            c_h = (c1_h, c2_h)[phase]
            w_h = (w1_h, w2_h)[phase]
            hp_h = (hp1_h, hp2_h)[phase]

            for e in range(_ZBLK):
                gbuf[0, e, pl.ds(0, 16)] = zero16
                gbuf[0, e, pl.ds(16, 16)] = zero16

            def zero_body(i, _):
                pltpu.async_copy(
                    gbuf.at[0, pl.ds(0, _ZBLK), :],
                    acc.at[pl.ds(wb_base + i * _ZBLK, _ZBLK), :],
                    esem.at[0])
                return 0
            lax.fori_loop(0, _WB_ROWS // _ZBLK, zero_body, 0)

            @pl.when(s == 0)
            def _():
                pltpu.async_copy(gbuf.at[0, pl.ds(0, _ZBLK), :],
                                 acc.at[pl.ds(tail_base, _ZBLK), :],
                                 esem.at[0])

            def zero_drain(i, _):
                pltpu.make_async_copy(
                    gbuf.at[0, pl.ds(0, _ZBLK), :],
                    acc.at[pl.ds(wb_base + i * _ZBLK, _ZBLK), :],
                    esem.at[0]).wait()
                return 0
            lax.fori_loop(0, _WB_ROWS // _ZBLK, zero_drain, 0)

            @pl.when(s == 0)
            def _():
                pltpu.make_async_copy(
                    gbuf.at[0, pl.ds(0, _ZBLK), :],
                    acc.at[pl.ds(tail_base, _ZBLK), :],
                    esem.at[0]).wait()

            plsc.subcore_barrier()

            def start_edges(ch):
                cb = ch % 2
                cid = s * _N_CHUNKS + ch
                pltpu.async_copy(r_h.at[cid], rowbuf.at[cb], esem.at[cb])
                pltpu.async_copy(c_h.at[cid], colbuf.at[cb], esem.at[cb])
                pltpu.async_copy(w_h.at[cid], wbuf.at[cb], esem.at[cb])

            def wait_edges(ch):
                cb = ch % 2
                cid = s * _N_CHUNKS + ch
                pltpu.make_async_copy(
                    r_h.at[cid], rowbuf.at[cb], esem.at[cb]).wait()
                pltpu.make_async_copy(
                    c_h.at[cid], colbuf.at[cb], esem.at[cb]).wait()
                pltpu.make_async_copy(
                    w_h.at[cid], wbuf.at[cb], esem.at[cb]).wait()

            def start_gather(cb, j):
                pltpu.async_copy(
                    h2_h.at[idxbuf.at[cb, j]], gbuf.at[j % _NB],
                    gsem.at[j % _NB])

            def wait_gather(cb, j):
                pltpu.make_async_copy(
                    h2_h.at[idxbuf.at[cb, j]], gbuf.at[j % _NB],
                    gsem.at[j % _NB]).wait()

            def start_scatter(cb, j):
                pltpu.async_copy(
                    gbuf.at[j % _NB], acc.at[rowbuf.at[cb, j]],
                    ssem.at[j % _NB], add=True)

            def wait_scatter(cb, j):
                pltpu.make_async_copy(
                    gbuf.at[j % _NB], acc.at[rowbuf.at[cb, j]],
                    ssem.at[j % _NB]).wait()

            start_edges(0)

            def chunk_body(ch, _):
                cb = ch % 2
                wait_edges(ch)

                @pl.when(ch < _N_CHUNKS - 1)
                def _():
                    start_edges(ch + 1)

                for j in range(_CHUNK_BLKS):
                    for q in range(_GBLK // 16):
                        v = colbuf[cb, j, pl.ds(q * 16, 16)]
                        idxbuf[cb, j, pl.ds(q * 16, 16)] = v + v + c

                start_gather(cb, 0)
                start_gather(cb, 1)
                start_gather(cb, 2)

                def blk_body(j, _):
                    jb = j % _NB
                    wait_gather(cb, j)

                    @pl.when(j < _CHUNK_BLKS - _NB + 1)
                    def _():
                        start_gather(cb, j + _NB - 1)

                    @pl.when(j >= _NB)
                    def _():
                        wait_scatter(cb, j - _NB)

                    for g in range(_GBLK // 16):
                        wv = wbuf[cb, j, pl.ds(g * 16, 16)]
                        for t in range(16):
                            e = g * 16 + t
                            ws = wv[t]
                            gbuf[jb, e, pl.ds(0, 16)] = (
                                gbuf[jb, e, pl.ds(0, 16)] * ws)
                            gbuf[jb, e, pl.ds(16, 16)] = (
                                gbuf[jb, e, pl.ds(16, 16)] * ws)
                    start_scatter(cb, j)
                    return 0
                lax.fori_loop(0, _CHUNK_BLKS, blk_body, 0)
                for j in range(_CHUNK_BLKS - _NB, _CHUNK_BLKS):
                    wait_scatter(cb, j)
                return 0
            lax.fori_loop(0, _N_CHUNKS, chunk_body, 0)

            plsc.subcore_barrier()

            pltpu.sync_copy(
                acc.at[pl.ds(wb_base, _WB_ROWS), :],
                hp_h.at[pl.ds(wb_base, _WB_ROWS), pl.ds(c * _HALF, _HALF)])

            @pl.when(s == 0)
            def _():
                pltpu.sync_copy(
                    acc.at[pl.ds(tail_base, _ZBLK), :],
                    hp_h.at[pl.ds(tail_base, _ZBLK), pl.ds(c * _HALF, _HALF)])

            plsc.subcore_barrier()

    return k(h2, r1, c1, w1, r2, c2, w2)


def kernel(seq_a, edge_index1, edge_weight1, edge_index2, edge_weight2,
           W1, b1, W2, b2):
    h_a = _mlp(seq_a, W1, b1, W2, b2)
    h2 = h_a.reshape(2 * _N, _HALF)
    pad = _E_PAD - _E

    def _shape_idx(x):
        return jnp.pad(x, (0, pad)).reshape(
            _N_CHUNKS_TOT, _CHUNK_BLKS, _GBLK)

    r1 = _shape_idx(edge_index1[0])
    c1 = _shape_idx(edge_index1[1])
    w1 = _shape_idx(edge_weight1)
    r2 = _shape_idx(edge_index2[0])
    c2 = _shape_idx(edge_index2[1])
    w2 = _shape_idx(edge_weight2)
    h_p1, h_p2 = _spmm_pair(h2, r1, c1, w1, r2, c2, w2)
    h_p_fusion = _fuse(h_p1, h_p2)
    return (h_a, h_p1, h_p2, h_p_fusion)
